# baseline (speedup 1.0000x reference)
import numpy as np
import jax
import jax.numpy as jnp
from jax.experimental import pallas as pl
from jax.experimental.pallas import tpu as pltpu

_N = 10000
_M = 2500
_K = 32
_K_UP = 3
_STRIDE = 4
_D = 128
_T = 128
_NT = 20
_KD = 544
_KF = 160

_BF = jnp.bfloat16
_TIN = 2000


def _shift(x, s):
    return jnp.concatenate([x[s:], x[:s]], axis=0)


def _win_mxu(x2500, band_ref, klen, step, acc_dtype):
    pieces = []
    for t in range(_NT):
        start = t * _T * step
        sl = x2500[start:start + klen]
        pieces.append(jax.lax.dot_general(
            band_ref[...], sl, (((1,), (0,)), ((), ())),
            preferred_element_type=jnp.float32).astype(acc_dtype))
    return jnp.concatenate(pieces, axis=0)


def _pad_wrap(x, total_rows, wrap_rows):
    z = jnp.zeros((total_rows - x.shape[0] - wrap_rows, _D), dtype=x.dtype)
    return jnp.concatenate([x, x[:wrap_rows], z], axis=0)


def _body(feat_ref, wd_ref, bd_ref, wf_ref, bf_ref, ws_ref, bs_ref,
          wm_ref, bm_ref, wu_ref, bu_ref, bdown_ref, b32_ref, out_ref,
          fvm_ref, sem_ref):
    f32 = jnp.float32

    def mm(a, b, out_dtype=f32):
        return jax.lax.dot_general(a.astype(_BF), b.astype(_BF),
                                   (((1,), (0,)), ((), ())),
                                   preferred_element_type=out_dtype)

    copies = []
    for t in range(_N // _TIN):
        cp = pltpu.make_async_copy(
            feat_ref.at[pl.ds(t * _TIN, _TIN), :],
            fvm_ref.at[pl.ds(t * _TIN, _TIN), :],
            sem_ref.at[t])
        cp.start()
        copies.append(cp)
    y0_tiles = []
    for t in range(_N // _TIN):
        copies[t].wait()
        y0_tiles.append(jnp.maximum(
            mm(fvm_ref[t * _TIN:(t + 1) * _TIN, :], wd_ref[...])
            + bd_ref[0, :], 0.0).astype(_BF))
    y0 = jnp.concatenate(y0_tiles, axis=0)
    y0p = _pad_wrap(y0, (_NT - 1) * _T * _STRIDE + _KD, _K - _STRIDE)
    feat1 = _win_mxu(y0p, bdown_ref, _KD, _STRIDE, _BF)

    y1 = jnp.maximum(mm(feat1, wf_ref[...]) + bf_ref[0, :], 0.0).astype(_BF)
    y1p = _pad_wrap(y1[:_M], (_NT - 1) * _T + _KF, _K)
    feat2 = _win_mxu(y1p, b32_ref, _KF, 1, _BF)

    ys = jnp.maximum(mm(feat2, ws_ref[...]) + bs_ref[0, :], 0.0).astype(_BF)
    ysp = _pad_wrap(ys[:_M], (_NT - 1) * _T + _KF, _K)
    skip = _win_mxu(ysp, b32_ref, _KF, 1, _BF)

    ym = jnp.maximum(mm(feat2, wm_ref[:_D]) + mm(skip, wm_ref[_D:])
                     + bm_ref[0, :], 0.0).astype(_BF)
    ymp = _pad_wrap(ym[:_M], (_NT - 1) * _T + _KF, _K)
    merge = _win_mxu(ymp, b32_ref, _KF, 1, f32)

    rows = jax.lax.broadcasted_iota(jnp.int32, (2 * _D, _D), 0)
    cols = jax.lax.broadcasted_iota(jnp.int32, (2 * _D, _D), 1)
    p = jnp.where(rows // 2 == cols, 1.0, 0.0).astype(f32)
    psum = mm(feat2, p[:_D]) + mm(skip, p[_D:])

    z = (mm(merge + psum, wu_ref[...]) + bu_ref[0, :])[:_M]
    u = (z + _shift(z, 1) + _shift(z, 2)) * (1.0 / _K_UP)

    out_ref[...] = jnp.broadcast_to(u[:, None, :],
                                    (_M, _STRIDE, _D)).reshape(_N, _D)


def _band(klen, step):
    i = np.arange(_T)[:, None]
    r = np.arange(klen)[None, :]
    b = ((r >= step * i) & (r <= step * i + _K - 1)).astype(np.float32) / _K
    return jnp.asarray(b, dtype=_BF)


def kernel(point_bcenter, point_feat, W_d0, b_d0, W_f0, b_f0, W_s, b_s,
           W_m, b_m, W_u, b_u):
    del point_bcenter
    args = (
        point_feat,
        W_d0, b_d0.reshape(1, _D),
        W_f0, b_f0.reshape(1, _D),
        W_s, b_s.reshape(1, _D),
        W_m, b_m.reshape(1, _D),
        W_u, b_u.reshape(1, _D),
        _band(_KD, _STRIDE), _band(_KF, 1),
    )
    vspec = pl.BlockSpec(memory_space=pltpu.MemorySpace.VMEM)
    in_specs = [pl.BlockSpec(memory_space=pltpu.MemorySpace.HBM)] + \
        [vspec] * 12
    out = pl.pallas_call(
        _body,
        in_specs=in_specs,
        out_specs=vspec,
        out_shape=jax.ShapeDtypeStruct((_N, _D), jnp.float32),
        scratch_shapes=[pltpu.VMEM((_N, _D), jnp.float32),
                        pltpu.SemaphoreType.DMA((_N // _TIN,))],
        compiler_params=pltpu.CompilerParams(
            vmem_limit_bytes=100 * 1024 * 1024),
    )(*args)
    return out


# speedup vs baseline: 1.0182x; 1.0182x over previous
import numpy as np
import jax
import jax.numpy as jnp
from jax.experimental import pallas as pl
from jax.experimental.pallas import tpu as pltpu

_N = 10000
_M = 2500
_K = 32
_K_UP = 3
_STRIDE = 4
_D = 128
_T = 128
_NT = 20
_KD = 544
_KF = 160

_BF = jnp.bfloat16


def _shift(x, s):
    return jnp.concatenate([x[s:], x[:s]], axis=0)


def _win_mxu(x2500, band_ref, klen, step, acc_dtype):
    pieces = []
    for t in range(_NT):
        start = t * _T * step
        sl = x2500[start:start + klen]
        pieces.append(jax.lax.dot_general(
            band_ref[...], sl, (((1,), (0,)), ((), ())),
            preferred_element_type=jnp.float32).astype(acc_dtype))
    return jnp.concatenate(pieces, axis=0)


def _pad_wrap(x, total_rows, wrap_rows):
    z = jnp.zeros((total_rows - x.shape[0] - wrap_rows, _D), dtype=x.dtype)
    return jnp.concatenate([x, x[:wrap_rows], z], axis=0)


def _body(feat_ref, wd_ref, bd_ref, wf_ref, bf_ref, ws_ref, bs_ref,
          wm_ref, bm_ref, wu_ref, bu_ref, bdown_ref, b32_ref, out_ref):
    f32 = jnp.float32

    def mm(a, b, out_dtype=f32):
        return jax.lax.dot_general(a.astype(_BF), b.astype(_BF),
                                   (((1,), (0,)), ((), ())),
                                   preferred_element_type=out_dtype)

    y0 = jnp.maximum(mm(feat_ref[...], wd_ref[...]) + bd_ref[0, :],
                     0.0).astype(_BF)
    y0p = _pad_wrap(y0, (_NT - 1) * _T * _STRIDE + _KD, _K - _STRIDE)
    feat1 = _win_mxu(y0p, bdown_ref, _KD, _STRIDE, _BF)

    y1 = jnp.maximum(mm(feat1, wf_ref[...]) + bf_ref[0, :], 0.0).astype(_BF)
    y1p = _pad_wrap(y1[:_M], (_NT - 1) * _T + _KF, _K)
    feat2 = _win_mxu(y1p, b32_ref, _KF, 1, _BF)

    ys = jnp.maximum(mm(feat2, ws_ref[...]) + bs_ref[0, :], 0.0).astype(_BF)
    ysp = _pad_wrap(ys[:_M], (_NT - 1) * _T + _KF, _K)
    skip = _win_mxu(ysp, b32_ref, _KF, 1, _BF)

    ym = jnp.maximum(mm(feat2, wm_ref[:_D]) + mm(skip, wm_ref[_D:])
                     + bm_ref[0, :], 0.0).astype(_BF)
    ymp = _pad_wrap(ym[:_M], (_NT - 1) * _T + _KF, _K)
    merge = _win_mxu(ymp, b32_ref, _KF, 1, f32)

    rows = jax.lax.broadcasted_iota(jnp.int32, (2 * _D, _D), 0)
    cols = jax.lax.broadcasted_iota(jnp.int32, (2 * _D, _D), 1)
    p = jnp.where(rows // 2 == cols, 1.0, 0.0).astype(f32)
    psum = mm(feat2, p[:_D]) + mm(skip, p[_D:])

    z = (mm(merge + psum, wu_ref[...]) + bu_ref[0, :])[:_M]
    u = (z + _shift(z, 1) + _shift(z, 2)) * (1.0 / _K_UP)

    out_ref[...] = jnp.broadcast_to(u[:, None, :],
                                    (_M, _STRIDE, _D)).reshape(_N, _D)


def _band(klen, step):
    i = np.arange(_T)[:, None]
    r = np.arange(klen)[None, :]
    b = ((r >= step * i) & (r <= step * i + _K - 1)).astype(np.float32) / _K
    return jnp.asarray(b, dtype=_BF)


def kernel(point_bcenter, point_feat, W_d0, b_d0, W_f0, b_f0, W_s, b_s,
           W_m, b_m, W_u, b_u):
    del point_bcenter
    args = (
        point_feat,
        W_d0, b_d0.reshape(1, _D),
        W_f0, b_f0.reshape(1, _D),
        W_s, b_s.reshape(1, _D),
        W_m, b_m.reshape(1, _D),
        W_u, b_u.reshape(1, _D),
        _band(_KD, _STRIDE), _band(_KF, 1),
    )
    out = pl.pallas_call(
        _body,
        out_shape=jax.ShapeDtypeStruct((_N, _D), jnp.float32),
        compiler_params=pltpu.CompilerParams(
            vmem_limit_bytes=100 * 1024 * 1024),
    )(*args)
    return out
